# baseline (speedup 1.0000x reference)
import functools

import jax
import jax.numpy as jnp
from jax import lax
from jax.experimental import pallas as pl
from jax.experimental.pallas import tpu as pltpu
from jax.experimental.pallas import tpu_sc as plsc

B, C, N, E, K = 4, 128, 10000, 320000, 3
NB = B * N
CH = C // 2
EB = 128
EPT = E // 16
NBF = EPT // EB
REM = EPT - NBF * EB
EPH = E // 32
NBH = EPH // EB
REMH = EPH - NBH * EB
NPAD = 10240
NSL = N // 16
F32 = jnp.float32
I32 = jnp.int32

_mesh = plsc.VectorSubcoreMesh(core_axis_name="c", subcore_axis_name="s")
_sc_params = pltpu.CompilerParams(needs_layout_passes=False,
                                  use_tc_tiling_on_sc=False)


def _rsqrt16(d):
    i = plsc.bitcast(d, I32)
    i = jnp.full((16,), 0x5F3759DF, I32) - lax.shift_right_arithmetic(i, 1)
    y = plsc.bitcast(i, F32)
    for _ in range(4):
        y = y * (1.5 - 0.5 * d * y * y)
    return y


def _norm_body(ei_h, ew_h, lw_h,
               pdeg2, pdeg1, rbuf, cbuf, wbuf, tmp, accb, dv, dvfull, lwbuf,
               sdeg, sdinv):
    c = lax.axis_index("c")
    s = lax.axis_index("s")
    lanes = lax.iota(I32, 16)
    rows8 = lanes & 7
    m_lo = lanes < 8
    m_hi = lanes >= 8
    z16 = jnp.zeros((16,), F32)

    def zero_pdeg(i, _):
        for r in range(8):
            pdeg2[pl.ds(r * NPAD + i * 16, 16)] = z16
        return 0
    lax.fori_loop(0, NPAD // 16, zero_pdeg, 0)

    def deg_step(eoff, n16):
        pltpu.sync_copy(ei_h.at[0, pl.ds(eoff, n16 * 16)], rbuf.at[pl.ds(0, n16 * 16)])
        pltpu.sync_copy(ei_h.at[1, pl.ds(eoff, n16 * 16)], cbuf.at[pl.ds(0, n16 * 16)])
        pltpu.sync_copy(ew_h.at[pl.ds(eoff, n16 * 16)], wbuf.at[pl.ds(0, n16 * 16)])
        for j in range(n16):
            r16 = rbuf[pl.ds(j * 16, 16)]
            c16 = cbuf[pl.ds(j * 16, 16)]
            w16 = wbuf[pl.ds(j * 16, 16)]
            w16 = jnp.where(r16 == c16, 0.0, w16)
            fidx = rows8 * NPAD + r16
            plsc.addupdate_scatter(pdeg2, [fidx], w16, mask=m_lo)
            plsc.addupdate_scatter(pdeg2, [fidx], w16, mask=m_hi)

    def deg_batch(k, _):
        deg_step(s * EPT + k * EB, EB // 16)
        return 0
    lax.fori_loop(0, NBF, deg_batch, 0)
    deg_step(s * EPT + NBF * EB, REM // 16)

    def red8(i, _):
        v = pdeg2[pl.ds(i * 16, 16)]
        for r in range(1, 8):
            v = v + pdeg2[pl.ds(r * NPAD + i * 16, 16)]
        pdeg1[pl.ds(i * 16, 16)] = v
        return 0
    lax.fori_loop(0, NPAD // 16, red8, 0)
    pltpu.sync_copy(pdeg1, sdeg.at[s])
    plsc.subcore_barrier()

    def zacc(i, _):
        accb[pl.ds(i * 16, 16)] = z16
        return 0
    lax.fori_loop(0, 40, zacc, 0)
    for j in range(16):
        pltpu.sync_copy(sdeg.at[j, pl.ds(s * 640, 640)], tmp)

        def addt(i, _):
            accb[pl.ds(i * 16, 16)] = accb[pl.ds(i * 16, 16)] + tmp[pl.ds(i * 16, 16)]
            return 0
        lax.fori_loop(0, 40, addt, 0)

    def dinv_chunk(i, _):
        d = accb[pl.ds(i * 16, 16)]
        y = _rsqrt16(d)
        dv[pl.ds(i * 16, 16)] = jnp.where(d > 0.0, y, 0.0)
        return 0
    lax.fori_loop(0, 40, dinv_chunk, 0)
    pltpu.sync_copy(dv, sdinv.at[pl.ds(s * 640, 640)])
    plsc.subcore_barrier()
    pltpu.sync_copy(sdinv, dvfull)

    base2 = c * (E // 2) + s * EPH

    def lw_step(eoff, n16):
        pltpu.sync_copy(ei_h.at[0, pl.ds(eoff, n16 * 16)], rbuf.at[pl.ds(0, n16 * 16)])
        pltpu.sync_copy(ei_h.at[1, pl.ds(eoff, n16 * 16)], cbuf.at[pl.ds(0, n16 * 16)])
        pltpu.sync_copy(ew_h.at[pl.ds(eoff, n16 * 16)], wbuf.at[pl.ds(0, n16 * 16)])
        for j in range(n16):
            r16 = rbuf[pl.ds(j * 16, 16)]
            c16 = cbuf[pl.ds(j * 16, 16)]
            w16 = wbuf[pl.ds(j * 16, 16)]
            w16 = jnp.where(r16 == c16, 0.0, w16)
            dr = plsc.load_gather(dvfull, [r16])
            dc = plsc.load_gather(dvfull, [c16])
            lwbuf[pl.ds(j * 16, 16)] = -(dr * w16 * dc)
        pltpu.sync_copy(lwbuf.at[pl.ds(0, n16 * 16)], lw_h.at[pl.ds(eoff, n16 * 16)])

    def lw_batch(k, _):
        lw_step(base2 + k * EB, EB // 16)
        return 0
    lax.fori_loop(0, NBH, lw_batch, 0)
    lw_step(base2 + NBH * EB, REMH // 16)


_norm_call = functools.partial(
    pl.kernel,
    out_type=jax.ShapeDtypeStruct((E,), F32),
    mesh=_mesh,
    scratch_types=[
        pltpu.VMEM((8 * NPAD,), F32),
        pltpu.VMEM((NPAD,), F32),
        pltpu.VMEM((EB,), I32),
        pltpu.VMEM((EB,), I32),
        pltpu.VMEM((EB,), F32),
        pltpu.VMEM((640,), F32),
        pltpu.VMEM((640,), F32),
        pltpu.VMEM((640,), F32),
        pltpu.VMEM((NPAD,), F32),
        pltpu.VMEM((EB,), F32),
        pltpu.VMEM_SHARED((16, NPAD), F32),
        pltpu.VMEM_SHARED((NPAD,), F32),
    ],
    compiler_params=_sc_params,
)(_norm_body)


def _prop_body(t_h, ei_h, lw_h, out_h,
               acc, rgf, cgf, lgf, ridx0, ridx1, cidx0, cidx1, ridxr, cidxr,
               gb0, gb1, gbr, sbuf, gs0, gs1, ss0, ss1):
    c = lax.axis_index("c")
    s = lax.axis_index("s")
    z16 = jnp.zeros((16,), F32)
    base_e = s * EPT

    pltpu.sync_copy(ei_h.at[0, pl.ds(base_e, EPT)], rgf)
    pltpu.sync_copy(ei_h.at[1, pl.ds(base_e, EPT)], cgf)
    pltpu.sync_copy(lw_h.at[pl.ds(base_e, EPT)], lgf)

    def zero_sbuf(i, _):
        for j in range(CH // 16):
            sbuf[i, pl.ds(j * 16, 16)] = z16
        return 0
    lax.fori_loop(0, EB, zero_sbuf, 0)

    def mkidx(ridx, cidx, k, off, n16):
        for j in range(n16):
            j2, jr = divmod(j, max(n16 // 2, 1))
            ridx[j2, pl.ds(jr * 16, 16)] = rgf[pl.ds(k * EB + j * 16, 16)] + off
            cidx[j2, pl.ds(jr * 16, 16)] = cgf[pl.ds(k * EB + j * 16, 16)]

    def scale_buf(gb, k, n16):
        def scale_grp(g, _):
            lw16 = lgf[pl.ds(k * EB + g * 16, 16)]
            for i in range(16):
                w = lw16[i]
                e = g * 16 + i
                for j in range(CH // 16):
                    gb[e, pl.ds(j * 16, 16)] = gb[e, pl.ds(j * 16, 16)] * w
            return 0
        lax.fori_loop(0, n16, scale_grp, 0)

    for bi in range(2):
        b = c * 2 + bi
        for h in range(2):
            off = h * NB + b * N

            for q in range(5):
                pltpu.sync_copy(sbuf.at[pl.ds(0, 125)],
                                acc.at[pl.ds(s * NSL + q * 125, 125)])
            plsc.subcore_barrier()

            def fire_gather(ridx, gb, sem):
                pltpu.async_copy(t_h.at[ridx.at[0]], gb.at[pl.ds(0, 64)], sem)
                pltpu.async_copy(t_h.at[ridx.at[1]], gb.at[pl.ds(64, 64)], sem)

            def drain_gather(ridx, gb, sem):
                pltpu.make_async_copy(t_h.at[ridx.at[0]],
                                      gb.at[pl.ds(0, 64)], sem).wait()
                pltpu.make_async_copy(t_h.at[ridx.at[1]],
                                      gb.at[pl.ds(64, 64)], sem).wait()

            def fire_scatter(cidx, gb, sem):
                pltpu.async_copy(gb.at[pl.ds(0, 64)],
                                 acc.at[cidx.at[0]], sem, add=True)
                pltpu.async_copy(gb.at[pl.ds(64, 64)],
                                 acc.at[cidx.at[1]], sem, add=True)

            def drain_scatter(cidx, gb, sem):
                pltpu.make_async_copy(gb.at[pl.ds(0, 64)],
                                      acc.at[cidx.at[0]], sem).wait()
                pltpu.make_async_copy(gb.at[pl.ds(64, 64)],
                                      acc.at[cidx.at[1]], sem).wait()

            mkidx(ridx0, cidx0, 0, off, 8)
            fire_gather(ridx0, gb0, gs0)
            mkidx(ridx1, cidx1, 1, off, 8)
            fire_gather(ridx1, gb1, gs1)

            def pair(kk, _):
                k0 = kk * 2
                k1 = k0 + 1
                drain_gather(ridx0, gb0, gs0)
                scale_buf(gb0, k0, 8)
                fire_scatter(cidx0, gb0, ss0)
                drain_gather(ridx1, gb1, gs1)
                scale_buf(gb1, k1, 8)
                fire_scatter(cidx1, gb1, ss1)
                drain_scatter(cidx0, gb0, ss0)

                @pl.when(kk < NBF // 2 - 1)
                def _():
                    mkidx(ridx0, cidx0, k0 + 2, off, 8)
                    fire_gather(ridx0, gb0, gs0)
                drain_scatter(cidx1, gb1, ss1)

                @pl.when(kk < NBF // 2 - 1)
                def _():
                    mkidx(ridx1, cidx1, k1 + 2, off, 8)
                    fire_gather(ridx1, gb1, gs1)
                return 0
            lax.fori_loop(0, NBF // 2, pair, 0)

            mkidx(ridxr, cidxr, NBF, off, REM // 16)
            pltpu.async_copy(t_h.at[ridxr.at[0]], gbr.at[pl.ds(0, 16)], gs0)
            pltpu.async_copy(t_h.at[ridxr.at[1]], gbr.at[pl.ds(16, 16)], gs0)
            pltpu.make_async_copy(t_h.at[ridxr.at[0]],
                                  gbr.at[pl.ds(0, 16)], gs0).wait()
            pltpu.make_async_copy(t_h.at[ridxr.at[1]],
                                  gbr.at[pl.ds(16, 16)], gs0).wait()
            scale_buf(gbr, NBF, REM // 16)
            pltpu.sync_copy(gbr.at[pl.ds(0, 16)], acc.at[cidxr.at[0]], add=True)
            pltpu.sync_copy(gbr.at[pl.ds(16, 16)], acc.at[cidxr.at[1]], add=True)
            plsc.subcore_barrier()

            for q in range(5):
                ro = s * NSL + q * 125
                pltpu.sync_copy(acc.at[pl.ds(ro, 125)], sbuf.at[pl.ds(0, 125)])
                pltpu.sync_copy(sbuf.at[pl.ds(0, 125)],
                                out_h.at[pl.ds(off + ro, 125)])
            if not (bi == 1 and h == 1):
                def rezero(i, _):
                    for j in range(CH // 16):
                        sbuf[i, pl.ds(j * 16, 16)] = z16
                    return 0
                lax.fori_loop(0, EB, rezero, 0)
                plsc.subcore_barrier()


_prop_call = functools.partial(
    pl.kernel,
    out_type=jax.ShapeDtypeStruct((2 * NB, CH), F32),
    mesh=_mesh,
    scratch_types=[
        pltpu.VMEM_SHARED((N, CH), F32),
        pltpu.VMEM((EPT,), I32),
        pltpu.VMEM((EPT,), I32),
        pltpu.VMEM((EPT,), F32),
        pltpu.VMEM((2, EB // 2), I32),
        pltpu.VMEM((2, EB // 2), I32),
        pltpu.VMEM((2, EB // 2), I32),
        pltpu.VMEM((2, EB // 2), I32),
        pltpu.VMEM((2, REM // 2), I32),
        pltpu.VMEM((2, REM // 2), I32),
        pltpu.VMEM((EB, CH), F32),
        pltpu.VMEM((EB, CH), F32),
        pltpu.VMEM((REM, CH), F32),
        pltpu.VMEM((EB, CH), F32),
        pltpu.SemaphoreType.DMA,
        pltpu.SemaphoreType.DMA,
        pltpu.SemaphoreType.DMA,
        pltpu.SemaphoreType.DMA,
    ],
    compiler_params=_sc_params,
)(_prop_body)


TN = 2000
NBLK = NB // TN


def _mm_body(x0_ref, x1_ref, a0_ref, a1_ref, b0_ref, b1_ref,
             w_ref, bb_ref, y_ref, s_ref):
    kidx = pl.program_id(0)
    u = jnp.concatenate(
        [x0_ref[...], x1_ref[...], a0_ref[...], a1_ref[...],
         2.0 * b0_ref[...], 2.0 * b1_ref[...]], axis=1)
    wc = jnp.concatenate([w_ref[0] - w_ref[2], w_ref[1], w_ref[2]], axis=0)
    yb = jnp.dot(u, wc, preferred_element_type=F32) + bb_ref[0:1, :]
    y_ref[...] = yb

    @pl.when(kidx == 0)
    def _():
        s_ref[...] = jnp.zeros((8, C), F32)
    s_ref[0:1, :] += jnp.sum(yb, axis=0, keepdims=True)
    s_ref[1:2, :] += jnp.sum(yb * yb, axis=0, keepdims=True)


def _bn_body(y_ref, sc_ref, sh_ref, o_ref):
    o_ref[...] = jnp.maximum(y_ref[...] * sc_ref[0:1, :] + sh_ref[0:1, :], 0.0)


def kernel(x, edge_index, edge_weight, W, b, gamma, beta):
    tsplit = (x.transpose(0, 2, 1)
              .reshape(NB, 2, CH).transpose(1, 0, 2).reshape(2 * NB, CH))

    lw = _norm_call(edge_index, edge_weight)
    t1 = _prop_call(tsplit, edge_index, lw)
    t2 = _prop_call(t1, edge_index, lw)

    bb = jnp.broadcast_to(b[None, :], (8, C))
    half = pl.BlockSpec((TN, CH), lambda k: (k, 0))
    half_hi = pl.BlockSpec((TN, CH), lambda k: (k + NBLK, 0))
    y, sums = pl.pallas_call(
        _mm_body,
        grid=(NBLK,),
        in_specs=[
            half, half_hi, half, half_hi, half, half_hi,
            pl.BlockSpec((K, C, C), lambda k: (0, 0, 0)),
            pl.BlockSpec((8, C), lambda k: (0, 0)),
        ],
        out_specs=[
            pl.BlockSpec((TN, C), lambda k: (k, 0)),
            pl.BlockSpec((8, C), lambda k: (0, 0)),
        ],
        out_shape=[
            jax.ShapeDtypeStruct((NB, C), F32),
            jax.ShapeDtypeStruct((8, C), F32),
        ],
    )(tsplit, tsplit, t1, t1, t2, t2, W, bb)

    mean = sums[0] / NB
    var = sums[1] / NB - mean * mean
    rstd = lax.rsqrt(var + 1e-5)
    scale = gamma * rstd
    shift = beta - mean * scale
    scb = jnp.broadcast_to(scale[None, :], (8, C))
    shb = jnp.broadcast_to(shift[None, :], (8, C))

    out = pl.pallas_call(
        _bn_body,
        grid=(NBLK,),
        in_specs=[
            pl.BlockSpec((TN, C), lambda k: (k, 0)),
            pl.BlockSpec((8, C), lambda k: (0, 0)),
            pl.BlockSpec((8, C), lambda k: (0, 0)),
        ],
        out_specs=pl.BlockSpec((TN, C), lambda k: (k, 0)),
        out_shape=jax.ShapeDtypeStruct((NB, C), F32),
    )(y, scb, shb)

    return out.reshape(B, N, C).transpose(0, 2, 1)


# speedup vs baseline: 1.3312x; 1.3312x over previous
import functools

import jax
import jax.numpy as jnp
from jax import lax
from jax.experimental import pallas as pl
from jax.experimental.pallas import tpu as pltpu
from jax.experimental.pallas import tpu_sc as plsc

B, C, N, E, K = 4, 128, 10000, 320000, 3
NB = B * N
CH = C // 2
EB = 128
EPT = E // 16
NBF = EPT // EB
REM = EPT - NBF * EB
EPH = E // 32
NBH = EPH // EB
REMH = EPH - NBH * EB
NPAD = 10240
NSL = N // 16
F32 = jnp.float32
I32 = jnp.int32

_mesh = plsc.VectorSubcoreMesh(core_axis_name="c", subcore_axis_name="s")
_sc_params = pltpu.CompilerParams(needs_layout_passes=False,
                                  use_tc_tiling_on_sc=False)


def _rsqrt16(d):
    i = plsc.bitcast(d, I32)
    i = jnp.full((16,), 0x5F3759DF, I32) - lax.shift_right_arithmetic(i, 1)
    y = plsc.bitcast(i, F32)
    for _ in range(4):
        y = y * (1.5 - 0.5 * d * y * y)
    return y


def _norm_body(ei_h, ew_h, lw_h, rc_h,
               pdeg2, pdeg1, rbuf, cbuf, wbuf, tmp, accb, dv, dvfull, lwbuf,
               rcbuf, sdeg, sdinv):
    c = lax.axis_index("c")
    s = lax.axis_index("s")
    lanes = lax.iota(I32, 16)
    rows8 = lanes & 7
    m_lo = lanes < 8
    m_hi = lanes >= 8
    z16 = jnp.zeros((16,), F32)

    def zero_pdeg(i, _):
        for r in range(8):
            pdeg2[pl.ds(r * NPAD + i * 16, 16)] = z16
        return 0
    lax.fori_loop(0, NPAD // 16, zero_pdeg, 0)

    def deg_step(eoff, n16):
        pltpu.sync_copy(ei_h.at[0, pl.ds(eoff, n16 * 16)], rbuf.at[pl.ds(0, n16 * 16)])
        pltpu.sync_copy(ei_h.at[1, pl.ds(eoff, n16 * 16)], cbuf.at[pl.ds(0, n16 * 16)])
        pltpu.sync_copy(ew_h.at[pl.ds(eoff, n16 * 16)], wbuf.at[pl.ds(0, n16 * 16)])
        for j in range(n16):
            r16 = rbuf[pl.ds(j * 16, 16)]
            c16 = cbuf[pl.ds(j * 16, 16)]
            w16 = wbuf[pl.ds(j * 16, 16)]
            w16 = jnp.where(r16 == c16, 0.0, w16)
            fidx = rows8 * NPAD + r16
            plsc.addupdate_scatter(pdeg2, [fidx], w16, mask=m_lo)
            plsc.addupdate_scatter(pdeg2, [fidx], w16, mask=m_hi)

    def deg_batch(k, _):
        deg_step(s * EPT + k * EB, EB // 16)
        return 0
    lax.fori_loop(0, NBF, deg_batch, 0)
    deg_step(s * EPT + NBF * EB, REM // 16)

    def red8(i, _):
        v = pdeg2[pl.ds(i * 16, 16)]
        for r in range(1, 8):
            v = v + pdeg2[pl.ds(r * NPAD + i * 16, 16)]
        pdeg1[pl.ds(i * 16, 16)] = v
        return 0
    lax.fori_loop(0, NPAD // 16, red8, 0)
    pltpu.sync_copy(pdeg1, sdeg.at[s])
    plsc.subcore_barrier()

    def zacc(i, _):
        accb[pl.ds(i * 16, 16)] = z16
        return 0
    lax.fori_loop(0, 40, zacc, 0)
    for j in range(16):
        pltpu.sync_copy(sdeg.at[j, pl.ds(s * 640, 640)], tmp)

        def addt(i, _):
            accb[pl.ds(i * 16, 16)] = accb[pl.ds(i * 16, 16)] + tmp[pl.ds(i * 16, 16)]
            return 0
        lax.fori_loop(0, 40, addt, 0)

    def dinv_chunk(i, _):
        d = accb[pl.ds(i * 16, 16)]
        y = _rsqrt16(d)
        dv[pl.ds(i * 16, 16)] = jnp.where(d > 0.0, y, 0.0)
        return 0
    lax.fori_loop(0, 40, dinv_chunk, 0)
    pltpu.sync_copy(dv, sdinv.at[pl.ds(s * 640, 640)])
    plsc.subcore_barrier()
    pltpu.sync_copy(sdinv, dvfull)

    base2 = jnp.where(c == 0, s * 10016, 160256 + s * 9984)

    def lw_step(eoff, n16):
        pltpu.sync_copy(ei_h.at[0, pl.ds(eoff, n16 * 16)], rbuf.at[pl.ds(0, n16 * 16)])
        pltpu.sync_copy(ei_h.at[1, pl.ds(eoff, n16 * 16)], cbuf.at[pl.ds(0, n16 * 16)])
        pltpu.sync_copy(ew_h.at[pl.ds(eoff, n16 * 16)], wbuf.at[pl.ds(0, n16 * 16)])

        def lw16(j):
            r16 = rbuf[pl.ds(j * 16, 16)]
            c16 = cbuf[pl.ds(j * 16, 16)]
            w16 = wbuf[pl.ds(j * 16, 16)]
            w16 = jnp.where(r16 == c16, 0.0, w16)
            dr = plsc.load_gather(dvfull, [r16])
            dc = plsc.load_gather(dvfull, [c16])
            rcbuf[pl.ds(j * 16, 16)] = lax.shift_left(r16, 14) | c16
            return -(dr * w16 * dc)

        for j2 in range(n16 // 2):
            ai = plsc.bitcast(lw16(2 * j2), I32)
            bi_ = plsc.bitcast(lw16(2 * j2 + 1), I32)
            vi = (lax.shift_right_logical(ai + 32768, 16)
                  | ((bi_ + 32768) & jnp.int32(-65536)))
            lwbuf[pl.ds(j2 * 32, 32)] = plsc.bitcast(vi, jnp.bfloat16)
        pltpu.sync_copy(lwbuf.at[pl.ds(0, n16 * 16)], lw_h.at[pl.ds(eoff, n16 * 16)])
        pltpu.sync_copy(rcbuf.at[pl.ds(0, n16 * 16)], rc_h.at[pl.ds(eoff, n16 * 16)])

    def lw_batch(k, _):
        lw_step(base2 + k * EB, EB // 16)
        return 0
    lax.fori_loop(0, NBH, lw_batch, 0)

    @pl.when(c == 0)
    def _():
        lw_step(base2 + NBH * EB, 2)


_norm_call = functools.partial(
    pl.kernel,
    out_type=[jax.ShapeDtypeStruct((E,), jnp.bfloat16),
              jax.ShapeDtypeStruct((E,), I32)],
    mesh=_mesh,
    scratch_types=[
        pltpu.VMEM((8 * NPAD,), F32),
        pltpu.VMEM((NPAD,), F32),
        pltpu.VMEM((EB,), I32),
        pltpu.VMEM((EB,), I32),
        pltpu.VMEM((EB,), F32),
        pltpu.VMEM((640,), F32),
        pltpu.VMEM((640,), F32),
        pltpu.VMEM((640,), F32),
        pltpu.VMEM((NPAD,), F32),
        pltpu.VMEM((EB,), jnp.bfloat16),
        pltpu.VMEM((EB,), I32),
        pltpu.VMEM_SHARED((16, NPAD), F32),
        pltpu.VMEM_SHARED((NPAD,), F32),
    ],
    compiler_params=_sc_params,
)(_norm_body)


def _prop_body(t_h, rc_h, lw_h, out_h,
               acc, tstage, rcf, lgf, ridx0, ridx1, cidx0, cidx1, ridxr, cidxr,
               gb0, gb1, gbr, gbh0, gbh1, gbhr, sbuf, gs0, gs1, ss0, ss1):
    c = lax.axis_index("c")
    s = lax.axis_index("s")
    z16 = jnp.zeros((16,), F32)
    base_e = s * EPT

    pltpu.sync_copy(rc_h.at[pl.ds(base_e, EPT)], rcf)
    pltpu.sync_copy(lw_h.at[pl.ds(base_e, EPT)], lgf)

    def zero_sbuf(i, _):
        for j in range(CH // 16):
            sbuf[i, pl.ds(j * 16, 16)] = z16
        return 0
    lax.fori_loop(0, EB, zero_sbuf, 0)

    def mkidx(ridx, cidx, k, off, n16):
        del off
        for j in range(n16):
            j2, jr = divmod(j, max(n16 // 2, 1))
            v = rcf[pl.ds(k * EB + j * 16, 16)]
            ridx[j2, pl.ds(jr * 16, 16)] = lax.shift_right_logical(v, 14)
            cidx[j2, pl.ds(jr * 16, 16)] = v & 16383

    def scale_buf(gbh, gb, k, n32):
        def scale_grp(g, _):
            v = plsc.bitcast(lgf[pl.ds(k * EB + g * 32, 32)], I32)
            wa = plsc.bitcast(lax.shift_left(v, 16), F32)
            wb = plsc.bitcast(v & jnp.int32(-65536), F32)
            for i in range(16):
                for (w, e) in ((wa[i], g * 32 + i), (wb[i], g * 32 + 16 + i)):
                    for j in range(CH // 32):
                        d = plsc.bitcast(gbh[e, pl.ds(j * 32, 32)], I32)
                        gb[e, pl.ds(j * 32, 16)] = (
                            plsc.bitcast(lax.shift_left(d, 16), F32) * w)
                        gb[e, pl.ds(j * 32 + 16, 16)] = (
                            plsc.bitcast(d & jnp.int32(-65536), F32) * w)
            return 0
        lax.fori_loop(0, n32, scale_grp, 0)

    for bi in range(2):
        b = c * 2 + bi
        for h in range(2):
            off = h * NB + b * N

            for q in range(5):
                pltpu.sync_copy(sbuf.at[pl.ds(0, 125)],
                                acc.at[pl.ds(s * NSL + q * 125, 125)])
            for q in range(5):
                ro = s * NSL + q * 125
                pltpu.sync_copy(t_h.at[pl.ds(off + ro, 125)],
                                gb0.at[pl.ds(0, 125)])

                def cvt_row(r, _):
                    for j in range(CH // 32):
                        ai = plsc.bitcast(gb0[r, pl.ds(j * 32, 16)], I32)
                        bi_ = plsc.bitcast(gb0[r, pl.ds(j * 32 + 16, 16)], I32)
                        vi = (lax.shift_right_logical(ai + 32768, 16)
                              | ((bi_ + 32768) & jnp.int32(-65536)))
                        gbh0[r, pl.ds(j * 32, 32)] = plsc.bitcast(vi, jnp.bfloat16)
                    return 0
                lax.fori_loop(0, 125, cvt_row, 0)
                pltpu.sync_copy(gbh0.at[pl.ds(0, 125)],
                                tstage.at[pl.ds(ro, 125)])
            plsc.subcore_barrier()

            def fire_gather(ridx, gbh, sem):
                pltpu.async_copy(tstage.at[ridx.at[0]], gbh.at[pl.ds(0, 64)], sem)
                pltpu.async_copy(tstage.at[ridx.at[1]], gbh.at[pl.ds(64, 64)], sem)

            def drain_gather(ridx, gbh, sem):
                pltpu.make_async_copy(tstage.at[ridx.at[0]],
                                      gbh.at[pl.ds(0, 64)], sem).wait()
                pltpu.make_async_copy(tstage.at[ridx.at[1]],
                                      gbh.at[pl.ds(64, 64)], sem).wait()

            def fire_scatter(cidx, gb, sem):
                pltpu.async_copy(gb.at[pl.ds(0, 64)],
                                 acc.at[cidx.at[0]], sem, add=True)
                pltpu.async_copy(gb.at[pl.ds(64, 64)],
                                 acc.at[cidx.at[1]], sem, add=True)

            def drain_scatter(cidx, gb, sem):
                pltpu.make_async_copy(gb.at[pl.ds(0, 64)],
                                      acc.at[cidx.at[0]], sem).wait()
                pltpu.make_async_copy(gb.at[pl.ds(64, 64)],
                                      acc.at[cidx.at[1]], sem).wait()

            mkidx(ridx0, cidx0, 0, off, 8)
            fire_gather(ridx0, gbh0, gs0)
            mkidx(ridx1, cidx1, 1, off, 8)
            fire_gather(ridx1, gbh1, gs1)

            def pair(kk, _):
                k0 = kk * 2
                k1 = k0 + 1
                drain_gather(ridx0, gbh0, gs0)
                scale_buf(gbh0, gb0, k0, 4)
                fire_scatter(cidx0, gb0, ss0)
                drain_gather(ridx1, gbh1, gs1)
                scale_buf(gbh1, gb1, k1, 4)
                fire_scatter(cidx1, gb1, ss1)
                drain_scatter(cidx0, gb0, ss0)

                @pl.when(kk < NBF // 2 - 1)
                def _():
                    mkidx(ridx0, cidx0, k0 + 2, off, 8)
                    fire_gather(ridx0, gbh0, gs0)
                drain_scatter(cidx1, gb1, ss1)

                @pl.when(kk < NBF // 2 - 1)
                def _():
                    mkidx(ridx1, cidx1, k1 + 2, off, 8)
                    fire_gather(ridx1, gbh1, gs1)
                return 0
            lax.fori_loop(0, NBF // 2, pair, 0)

            mkidx(ridxr, cidxr, NBF, off, REM // 16)
            pltpu.async_copy(tstage.at[ridxr.at[0]], gbhr.at[pl.ds(0, 16)], gs0)
            pltpu.async_copy(tstage.at[ridxr.at[1]], gbhr.at[pl.ds(16, 16)], gs0)
            pltpu.make_async_copy(tstage.at[ridxr.at[0]],
                                  gbhr.at[pl.ds(0, 16)], gs0).wait()
            pltpu.make_async_copy(tstage.at[ridxr.at[1]],
                                  gbhr.at[pl.ds(16, 16)], gs0).wait()
            scale_buf(gbhr, gbr, NBF, REM // 32)
            pltpu.sync_copy(gbr.at[pl.ds(0, 16)], acc.at[cidxr.at[0]], add=True)
            pltpu.sync_copy(gbr.at[pl.ds(16, 16)], acc.at[cidxr.at[1]], add=True)
            plsc.subcore_barrier()

            for q in range(5):
                ro = s * NSL + q * 125
                pltpu.sync_copy(acc.at[pl.ds(ro, 125)], sbuf.at[pl.ds(0, 125)])
                pltpu.sync_copy(sbuf.at[pl.ds(0, 125)],
                                out_h.at[pl.ds(off + ro, 125)])
            if not (bi == 1 and h == 1):
                def rezero(i, _):
                    for j in range(CH // 16):
                        sbuf[i, pl.ds(j * 16, 16)] = z16
                    return 0
                lax.fori_loop(0, EB, rezero, 0)
                plsc.subcore_barrier()


_prop_call = functools.partial(
    pl.kernel,
    out_type=jax.ShapeDtypeStruct((2 * NB, CH), F32),
    mesh=_mesh,
    scratch_types=[
        pltpu.VMEM_SHARED((N, CH), F32),
        pltpu.VMEM_SHARED((N, CH), jnp.bfloat16),
        pltpu.VMEM((EPT,), I32),
        pltpu.VMEM((EPT,), jnp.bfloat16),
        pltpu.VMEM((2, EB // 2), I32),
        pltpu.VMEM((2, EB // 2), I32),
        pltpu.VMEM((2, EB // 2), I32),
        pltpu.VMEM((2, EB // 2), I32),
        pltpu.VMEM((2, REM // 2), I32),
        pltpu.VMEM((2, REM // 2), I32),
        pltpu.VMEM((EB, CH), F32),
        pltpu.VMEM((EB, CH), F32),
        pltpu.VMEM((REM, CH), F32),
        pltpu.VMEM((EB, CH), jnp.bfloat16),
        pltpu.VMEM((EB, CH), jnp.bfloat16),
        pltpu.VMEM((REM, CH), jnp.bfloat16),
        pltpu.VMEM((EB, CH), F32),
        pltpu.SemaphoreType.DMA,
        pltpu.SemaphoreType.DMA,
        pltpu.SemaphoreType.DMA,
        pltpu.SemaphoreType.DMA,
    ],
    compiler_params=_sc_params,
)(_prop_body)


TN = 2000
NBLK = NB // TN


def _mm_body(x0_ref, x1_ref, a0_ref, a1_ref, b0_ref, b1_ref,
             w_ref, bb_ref, y_ref, s_ref):
    kidx = pl.program_id(0)
    u = jnp.concatenate(
        [x0_ref[...], x1_ref[...], a0_ref[...], a1_ref[...],
         2.0 * b0_ref[...], 2.0 * b1_ref[...]], axis=1)
    wc = jnp.concatenate([w_ref[0] - w_ref[2], w_ref[1], w_ref[2]], axis=0)
    yb = jnp.dot(u, wc, preferred_element_type=F32) + bb_ref[0:1, :]
    y_ref[...] = yb

    @pl.when(kidx == 0)
    def _():
        s_ref[...] = jnp.zeros((8, C), F32)
    s_ref[0:1, :] += jnp.sum(yb, axis=0, keepdims=True)
    s_ref[1:2, :] += jnp.sum(yb * yb, axis=0, keepdims=True)


def _bn_body(y_ref, sc_ref, sh_ref, o_ref):
    o_ref[...] = jnp.maximum(y_ref[...] * sc_ref[0:1, :] + sh_ref[0:1, :], 0.0)


def kernel(x, edge_index, edge_weight, W, b, gamma, beta):
    tsplit = (x.transpose(0, 2, 1)
              .reshape(NB, 2, CH).transpose(1, 0, 2).reshape(2 * NB, CH))

    lw, rc = _norm_call(edge_index, edge_weight)
    t1 = _prop_call(tsplit, rc, lw)
    t2 = _prop_call(t1, rc, lw)

    bb = jnp.broadcast_to(b[None, :], (8, C))
    half = pl.BlockSpec((TN, CH), lambda k: (k, 0))
    half_hi = pl.BlockSpec((TN, CH), lambda k: (k + NBLK, 0))
    y, sums = pl.pallas_call(
        _mm_body,
        grid=(NBLK,),
        in_specs=[
            half, half_hi, half, half_hi, half, half_hi,
            pl.BlockSpec((K, C, C), lambda k: (0, 0, 0)),
            pl.BlockSpec((8, C), lambda k: (0, 0)),
        ],
        out_specs=[
            pl.BlockSpec((TN, C), lambda k: (k, 0)),
            pl.BlockSpec((8, C), lambda k: (0, 0)),
        ],
        out_shape=[
            jax.ShapeDtypeStruct((NB, C), F32),
            jax.ShapeDtypeStruct((8, C), F32),
        ],
    )(tsplit, tsplit, t1, t1, t2, t2, W, bb)

    mean = sums[0] / NB
    var = sums[1] / NB - mean * mean
    rstd = lax.rsqrt(var + 1e-5)
    scale = gamma * rstd
    shift = beta - mean * scale
    scb = jnp.broadcast_to(scale[None, :], (8, C))
    shb = jnp.broadcast_to(shift[None, :], (8, C))

    out = pl.pallas_call(
        _bn_body,
        grid=(NBLK,),
        in_specs=[
            pl.BlockSpec((TN, C), lambda k: (k, 0)),
            pl.BlockSpec((8, C), lambda k: (0, 0)),
            pl.BlockSpec((8, C), lambda k: (0, 0)),
        ],
        out_specs=pl.BlockSpec((TN, C), lambda k: (k, 0)),
        out_shape=jax.ShapeDtypeStruct((NB, C), F32),
    )(y, scb, shb)

    return out.reshape(B, N, C).transpose(0, 2, 1)


# speedup vs baseline: 1.4418x; 1.0831x over previous
import functools

import jax
import jax.numpy as jnp
from jax import lax
from jax.experimental import pallas as pl
from jax.experimental.pallas import tpu as pltpu
from jax.experimental.pallas import tpu_sc as plsc

B, C, N, E, K = 4, 128, 10000, 320000, 3
NB = B * N
CH = C // 2
EB = 128
EPT = E // 16
NBF = EPT // EB
REM = EPT - NBF * EB
EPH = E // 32
NBH = EPH // EB
REMH = EPH - NBH * EB
NPAD = 10240
NSL = N // 16
F32 = jnp.float32
I32 = jnp.int32

_mesh = plsc.VectorSubcoreMesh(core_axis_name="c", subcore_axis_name="s")
_sc_params = pltpu.CompilerParams(needs_layout_passes=False,
                                  use_tc_tiling_on_sc=False)


def _rsqrt16(d):
    i = plsc.bitcast(d, I32)
    i = jnp.full((16,), 0x5F3759DF, I32) - lax.shift_right_arithmetic(i, 1)
    y = plsc.bitcast(i, F32)
    for _ in range(4):
        y = y * (1.5 - 0.5 * d * y * y)
    return y


def _norm_body(ei_h, ew_h, lw_h, rc_h,
               pdeg2, pdeg1, rbuf, cbuf, wbuf, tmp, accb, dv, dvfull, lwbuf,
               rcbuf, sdeg, sdinv):
    c = lax.axis_index("c")
    s = lax.axis_index("s")
    lanes = lax.iota(I32, 16)
    rows8 = lanes & 7
    m_lo = lanes < 8
    m_hi = lanes >= 8
    z16 = jnp.zeros((16,), F32)

    def zero_pdeg(i, _):
        for r in range(8):
            pdeg2[pl.ds(r * NPAD + i * 16, 16)] = z16
        return 0
    lax.fori_loop(0, NPAD // 16, zero_pdeg, 0)

    def deg_step(eoff, n16):
        pltpu.sync_copy(ei_h.at[0, pl.ds(eoff, n16 * 16)], rbuf.at[pl.ds(0, n16 * 16)])
        pltpu.sync_copy(ei_h.at[1, pl.ds(eoff, n16 * 16)], cbuf.at[pl.ds(0, n16 * 16)])
        pltpu.sync_copy(ew_h.at[pl.ds(eoff, n16 * 16)], wbuf.at[pl.ds(0, n16 * 16)])
        for j in range(n16):
            r16 = rbuf[pl.ds(j * 16, 16)]
            c16 = cbuf[pl.ds(j * 16, 16)]
            w16 = wbuf[pl.ds(j * 16, 16)]
            w16 = jnp.where(r16 == c16, 0.0, w16)
            fidx = rows8 * NPAD + r16
            plsc.addupdate_scatter(pdeg2, [fidx], w16, mask=m_lo)
            plsc.addupdate_scatter(pdeg2, [fidx], w16, mask=m_hi)

    def deg_batch(k, _):
        deg_step(s * EPT + k * 1024, 64)
        return 0
    lax.fori_loop(0, 19, deg_batch, 0)
    deg_step(s * EPT + 19 * 1024, 34)

    def red8(i, _):
        v = pdeg2[pl.ds(i * 16, 16)]
        for r in range(1, 8):
            v = v + pdeg2[pl.ds(r * NPAD + i * 16, 16)]
        pdeg1[pl.ds(i * 16, 16)] = v
        return 0
    lax.fori_loop(0, NPAD // 16, red8, 0)
    pltpu.sync_copy(pdeg1, sdeg.at[s])
    plsc.subcore_barrier()

    def zacc(i, _):
        accb[pl.ds(i * 16, 16)] = z16
        return 0
    lax.fori_loop(0, 40, zacc, 0)
    for j in range(16):
        pltpu.sync_copy(sdeg.at[j, pl.ds(s * 640, 640)], tmp)

        def addt(i, _):
            accb[pl.ds(i * 16, 16)] = accb[pl.ds(i * 16, 16)] + tmp[pl.ds(i * 16, 16)]
            return 0
        lax.fori_loop(0, 40, addt, 0)

    def dinv_chunk(i, _):
        d = accb[pl.ds(i * 16, 16)]
        y = _rsqrt16(d)
        dv[pl.ds(i * 16, 16)] = jnp.where(d > 0.0, y, 0.0)
        return 0
    lax.fori_loop(0, 40, dinv_chunk, 0)
    pltpu.sync_copy(dv, sdinv.at[pl.ds(s * 640, 640)])
    plsc.subcore_barrier()
    pltpu.sync_copy(sdinv, dvfull)

    base2 = jnp.where(c == 0, s * 10016, 160256 + s * 9984)

    def lw_step(eoff, n16):
        pltpu.sync_copy(ei_h.at[0, pl.ds(eoff, n16 * 16)], rbuf.at[pl.ds(0, n16 * 16)])
        pltpu.sync_copy(ei_h.at[1, pl.ds(eoff, n16 * 16)], cbuf.at[pl.ds(0, n16 * 16)])
        pltpu.sync_copy(ew_h.at[pl.ds(eoff, n16 * 16)], wbuf.at[pl.ds(0, n16 * 16)])

        def lw16(j):
            r16 = rbuf[pl.ds(j * 16, 16)]
            c16 = cbuf[pl.ds(j * 16, 16)]
            w16 = wbuf[pl.ds(j * 16, 16)]
            w16 = jnp.where(r16 == c16, 0.0, w16)
            dr = plsc.load_gather(dvfull, [r16])
            dc = plsc.load_gather(dvfull, [c16])
            rcbuf[pl.ds(j * 16, 16)] = lax.shift_left(r16, 14) | c16
            return -(dr * w16 * dc)

        for j2 in range(n16 // 2):
            ai = plsc.bitcast(lw16(2 * j2), I32)
            bi_ = plsc.bitcast(lw16(2 * j2 + 1), I32)
            vi = (lax.shift_right_logical(ai + 32768, 16)
                  | ((bi_ + 32768) & jnp.int32(-65536)))
            lwbuf[pl.ds(j2 * 32, 32)] = plsc.bitcast(vi, jnp.bfloat16)
        pltpu.sync_copy(lwbuf.at[pl.ds(0, n16 * 16)], lw_h.at[pl.ds(eoff, n16 * 16)])
        pltpu.sync_copy(rcbuf.at[pl.ds(0, n16 * 16)], rc_h.at[pl.ds(eoff, n16 * 16)])

    def lw_batch(k, _):
        lw_step(base2 + k * 1024, 64)
        return 0
    lax.fori_loop(0, 9, lw_batch, 0)
    lw_step(base2 + 9 * 1024, 48)

    @pl.when(c == 0)
    def _():
        lw_step(base2 + NBH * EB, 2)


_norm_call = functools.partial(
    pl.kernel,
    out_type=[jax.ShapeDtypeStruct((E,), jnp.bfloat16),
              jax.ShapeDtypeStruct((E,), I32)],
    mesh=_mesh,
    scratch_types=[
        pltpu.VMEM((8 * NPAD,), F32),
        pltpu.VMEM((NPAD,), F32),
        pltpu.VMEM((1024,), I32),
        pltpu.VMEM((1024,), I32),
        pltpu.VMEM((1024,), F32),
        pltpu.VMEM((640,), F32),
        pltpu.VMEM((640,), F32),
        pltpu.VMEM((640,), F32),
        pltpu.VMEM((NPAD,), F32),
        pltpu.VMEM((1024,), jnp.bfloat16),
        pltpu.VMEM((1024,), I32),
        pltpu.VMEM_SHARED((16, NPAD), F32),
        pltpu.VMEM_SHARED((NPAD,), F32),
    ],
    compiler_params=_sc_params,
)(_norm_body)


def _prop_body(t_h, rc_h, lw_h, out_h,
               acc, tstage, rcf, lgf, ridx0, ridx1, cidx0, cidx1, ridxr, cidxr,
               gb0, gb1, gbr, gbh0, gbh1, gbhr, sbuf, gs0, gs1, ss0, ss1):
    c = lax.axis_index("c")
    s = lax.axis_index("s")
    z16 = jnp.zeros((16,), F32)
    base_e = s * EPT

    pltpu.sync_copy(rc_h.at[pl.ds(base_e, EPT)], rcf)
    pltpu.sync_copy(lw_h.at[pl.ds(base_e, EPT)], lgf)

    def zero_sbuf(i, _):
        for j in range(CH // 16):
            sbuf[i, pl.ds(j * 16, 16)] = z16
        return 0
    lax.fori_loop(0, EB, zero_sbuf, 0)

    def mkidx(ridx, cidx, k, off, n16):
        del off
        for j in range(n16):
            j2, jr = divmod(j, max(n16 // 2, 1))
            v = rcf[pl.ds(k * EB + j * 16, 16)]
            ridx[j2, pl.ds(jr * 16, 16)] = lax.shift_right_logical(v, 14)
            cidx[j2, pl.ds(jr * 16, 16)] = v & 16383

    def scale_buf(gbh, gb, k, n32):
        def scale_grp(g, _):
            v = plsc.bitcast(lgf[pl.ds(k * EB + g * 32, 32)], I32)
            wa = plsc.bitcast(lax.shift_left(v, 16), F32)
            wb = plsc.bitcast(v & jnp.int32(-65536), F32)
            for i in range(16):
                for (w, e) in ((wa[i], g * 32 + i), (wb[i], g * 32 + 16 + i)):
                    for j in range(CH // 32):
                        d = plsc.bitcast(gbh[e, pl.ds(j * 32, 32)], I32)
                        gb[e, pl.ds(j * 32, 16)] = (
                            plsc.bitcast(lax.shift_left(d, 16), F32) * w)
                        gb[e, pl.ds(j * 32 + 16, 16)] = (
                            plsc.bitcast(d & jnp.int32(-65536), F32) * w)
            return 0
        lax.fori_loop(0, n32, scale_grp, 0)

    for bi in range(2):
        b = c * 2 + bi
        for h in range(2):
            off = h * NB + b * N

            for q in range(5):
                pltpu.sync_copy(sbuf.at[pl.ds(0, 125)],
                                acc.at[pl.ds(s * NSL + q * 125, 125)])
            for q in range(5):
                ro = s * NSL + q * 125
                pltpu.sync_copy(t_h.at[pl.ds(off + ro, 125)],
                                gbh0.at[pl.ds(0, 125)])
                pltpu.sync_copy(gbh0.at[pl.ds(0, 125)],
                                tstage.at[pl.ds(ro, 125)])
            plsc.subcore_barrier()

            def fire_gather(ridx, gbh, sem):
                pltpu.async_copy(tstage.at[ridx.at[0]], gbh.at[pl.ds(0, 64)], sem)
                pltpu.async_copy(tstage.at[ridx.at[1]], gbh.at[pl.ds(64, 64)], sem)

            def drain_gather(ridx, gbh, sem):
                pltpu.make_async_copy(tstage.at[ridx.at[0]],
                                      gbh.at[pl.ds(0, 64)], sem).wait()
                pltpu.make_async_copy(tstage.at[ridx.at[1]],
                                      gbh.at[pl.ds(64, 64)], sem).wait()

            def fire_scatter(cidx, gb, sem):
                pltpu.async_copy(gb.at[pl.ds(0, 64)],
                                 acc.at[cidx.at[0]], sem, add=True)
                pltpu.async_copy(gb.at[pl.ds(64, 64)],
                                 acc.at[cidx.at[1]], sem, add=True)

            def drain_scatter(cidx, gb, sem):
                pltpu.make_async_copy(gb.at[pl.ds(0, 64)],
                                      acc.at[cidx.at[0]], sem).wait()
                pltpu.make_async_copy(gb.at[pl.ds(64, 64)],
                                      acc.at[cidx.at[1]], sem).wait()

            mkidx(ridx0, cidx0, 0, off, 8)
            fire_gather(ridx0, gbh0, gs0)
            mkidx(ridx1, cidx1, 1, off, 8)
            fire_gather(ridx1, gbh1, gs1)

            def pair(kk, _):
                k0 = kk * 2
                k1 = k0 + 1
                drain_gather(ridx0, gbh0, gs0)
                scale_buf(gbh0, gb0, k0, 4)
                fire_scatter(cidx0, gb0, ss0)
                drain_gather(ridx1, gbh1, gs1)
                scale_buf(gbh1, gb1, k1, 4)
                fire_scatter(cidx1, gb1, ss1)
                drain_scatter(cidx0, gb0, ss0)

                @pl.when(kk < NBF // 2 - 1)
                def _():
                    mkidx(ridx0, cidx0, k0 + 2, off, 8)
                    fire_gather(ridx0, gbh0, gs0)
                drain_scatter(cidx1, gb1, ss1)

                @pl.when(kk < NBF // 2 - 1)
                def _():
                    mkidx(ridx1, cidx1, k1 + 2, off, 8)
                    fire_gather(ridx1, gbh1, gs1)
                return 0
            lax.fori_loop(0, NBF // 2, pair, 0)

            mkidx(ridxr, cidxr, NBF, off, REM // 16)
            pltpu.async_copy(tstage.at[ridxr.at[0]], gbhr.at[pl.ds(0, 16)], gs0)
            pltpu.async_copy(tstage.at[ridxr.at[1]], gbhr.at[pl.ds(16, 16)], gs0)
            pltpu.make_async_copy(tstage.at[ridxr.at[0]],
                                  gbhr.at[pl.ds(0, 16)], gs0).wait()
            pltpu.make_async_copy(tstage.at[ridxr.at[1]],
                                  gbhr.at[pl.ds(16, 16)], gs0).wait()
            scale_buf(gbhr, gbr, NBF, REM // 32)
            pltpu.sync_copy(gbr.at[pl.ds(0, 16)], acc.at[cidxr.at[0]], add=True)
            pltpu.sync_copy(gbr.at[pl.ds(16, 16)], acc.at[cidxr.at[1]], add=True)
            plsc.subcore_barrier()

            for q in range(5):
                ro = s * NSL + q * 125
                pltpu.sync_copy(acc.at[pl.ds(ro, 125)], sbuf.at[pl.ds(0, 125)])

                def cvt_row(r, _):
                    for j in range(CH // 32):
                        ai = plsc.bitcast(sbuf[r, pl.ds(j * 32, 16)], I32)
                        bi_ = plsc.bitcast(sbuf[r, pl.ds(j * 32 + 16, 16)], I32)
                        vi = (lax.shift_right_logical(ai + 32768, 16)
                              | ((bi_ + 32768) & jnp.int32(-65536)))
                        gbh0[r, pl.ds(j * 32, 32)] = plsc.bitcast(vi, jnp.bfloat16)
                    return 0
                lax.fori_loop(0, 125, cvt_row, 0)
                pltpu.sync_copy(gbh0.at[pl.ds(0, 125)],
                                out_h.at[pl.ds(off + ro, 125)])
            if not (bi == 1 and h == 1):
                def rezero(i, _):
                    for j in range(CH // 16):
                        sbuf[i, pl.ds(j * 16, 16)] = z16
                    return 0
                lax.fori_loop(0, EB, rezero, 0)
                plsc.subcore_barrier()


_prop_call = functools.partial(
    pl.kernel,
    out_type=jax.ShapeDtypeStruct((2 * NB, CH), jnp.bfloat16),
    mesh=_mesh,
    scratch_types=[
        pltpu.VMEM_SHARED((N, CH), F32),
        pltpu.VMEM_SHARED((N, CH), jnp.bfloat16),
        pltpu.VMEM((EPT,), I32),
        pltpu.VMEM((EPT,), jnp.bfloat16),
        pltpu.VMEM((2, EB // 2), I32),
        pltpu.VMEM((2, EB // 2), I32),
        pltpu.VMEM((2, EB // 2), I32),
        pltpu.VMEM((2, EB // 2), I32),
        pltpu.VMEM((2, REM // 2), I32),
        pltpu.VMEM((2, REM // 2), I32),
        pltpu.VMEM((EB, CH), F32),
        pltpu.VMEM((EB, CH), F32),
        pltpu.VMEM((REM, CH), F32),
        pltpu.VMEM((EB, CH), jnp.bfloat16),
        pltpu.VMEM((EB, CH), jnp.bfloat16),
        pltpu.VMEM((REM, CH), jnp.bfloat16),
        pltpu.VMEM((EB, CH), F32),
        pltpu.SemaphoreType.DMA,
        pltpu.SemaphoreType.DMA,
        pltpu.SemaphoreType.DMA,
        pltpu.SemaphoreType.DMA,
    ],
    compiler_params=_sc_params,
)(_prop_body)


TN = 2000
NBLK = NB // TN


def _mm_body(x0_ref, x1_ref, a0_ref, a1_ref, b0_ref, b1_ref,
             w_ref, bb_ref, y_ref, s_ref):
    kidx = pl.program_id(0)
    two = jnp.bfloat16(2.0)
    u = jnp.concatenate(
        [x0_ref[...], x1_ref[...], a0_ref[...], a1_ref[...],
         two * b0_ref[...], two * b1_ref[...]], axis=1)
    wc = jnp.concatenate([w_ref[0] - w_ref[2], w_ref[1], w_ref[2]],
                         axis=0).astype(jnp.bfloat16)
    yb = jnp.dot(u, wc, preferred_element_type=F32) + bb_ref[0:1, :]
    y_ref[...] = yb

    @pl.when(kidx == 0)
    def _():
        s_ref[...] = jnp.zeros((8, C), F32)
    s_ref[0:1, :] += jnp.sum(yb, axis=0, keepdims=True)
    s_ref[1:2, :] += jnp.sum(yb * yb, axis=0, keepdims=True)


def _bn_body(y_ref, sc_ref, sh_ref, o_ref):
    o_ref[...] = jnp.maximum(y_ref[...] * sc_ref[0:1, :] + sh_ref[0:1, :], 0.0)


def kernel(x, edge_index, edge_weight, W, b, gamma, beta):
    tsplit = (x.transpose(0, 2, 1)
              .reshape(NB, 2, CH).transpose(1, 0, 2).reshape(2 * NB, CH)
              .astype(jnp.bfloat16))

    lw, rc = _norm_call(edge_index, edge_weight)
    t1 = _prop_call(tsplit, rc, lw)
    t2 = _prop_call(t1, rc, lw)

    bb = jnp.broadcast_to(b[None, :], (8, C))
    half = pl.BlockSpec((TN, CH), lambda k: (k, 0))
    half_hi = pl.BlockSpec((TN, CH), lambda k: (k + NBLK, 0))
    y, sums = pl.pallas_call(
        _mm_body,
        grid=(NBLK,),
        in_specs=[
            half, half_hi, half, half_hi, half, half_hi,
            pl.BlockSpec((K, C, C), lambda k: (0, 0, 0)),
            pl.BlockSpec((8, C), lambda k: (0, 0)),
        ],
        out_specs=[
            pl.BlockSpec((TN, C), lambda k: (k, 0)),
            pl.BlockSpec((8, C), lambda k: (0, 0)),
        ],
        out_shape=[
            jax.ShapeDtypeStruct((NB, C), F32),
            jax.ShapeDtypeStruct((8, C), F32),
        ],
    )(tsplit, tsplit, t1, t1, t2, t2, W, bb)

    mean = sums[0] / NB
    var = sums[1] / NB - mean * mean
    rstd = lax.rsqrt(var + 1e-5)
    scale = gamma * rstd
    shift = beta - mean * scale
    scb = jnp.broadcast_to(scale[None, :], (8, C))
    shb = jnp.broadcast_to(shift[None, :], (8, C))

    out = pl.pallas_call(
        _bn_body,
        grid=(NBLK,),
        in_specs=[
            pl.BlockSpec((TN, C), lambda k: (k, 0)),
            pl.BlockSpec((8, C), lambda k: (0, 0)),
            pl.BlockSpec((8, C), lambda k: (0, 0)),
        ],
        out_specs=pl.BlockSpec((TN, C), lambda k: (k, 0)),
        out_shape=jax.ShapeDtypeStruct((NB, C), F32),
    )(y, scb, shb)

    return out.reshape(B, N, C).transpose(0, 2, 1)
